# Initial kernel scaffold; baseline (speedup 1.0000x reference)
import jax
import jax.numpy as jnp
from jax.experimental import pallas as pl


def kernel(node_feat, node_type_feat, query, edge_index, spans, seq_len, W, a_src, a_dst, Wq, Wk, Wv):
    raise NotImplementedError("write your pallas kernel here")


# speedup vs baseline: 30.8452x; 30.8452x over previous
import functools

import jax
import jax.numpy as jnp
from jax import lax
from jax.experimental import pallas as pl
from jax.experimental.pallas import tpu as pltpu
from jax.experimental.pallas import tpu_sc as plsc

_B, _G, _NPER, _EPER, _H, _S = 8, 16, 64, 1024, 512, 2048
_NG = _B * _G
_NC, _NS = 2, 16
_NW = _NC * _NS
_GPW = _NG // _NW
_L = 16


def _proj_body(nf_ref, nt_ref, w_ref, a2_ref, h_ref, hsd_ref):
    dn = nf_ref.shape[1]
    h = jnp.dot(nf_ref[...], w_ref[:dn, :], preferred_element_type=jnp.float32)
    h = h + jnp.dot(nt_ref[...], w_ref[dn:, :], preferred_element_type=jnp.float32)
    h_ref[...] = h
    hsd_ref[...] = jnp.dot(h, a2_ref[...], preferred_element_type=jnp.float32)


@functools.cache
def _make_edge_kernel():
    mesh = plsc.VectorSubcoreMesh(core_axis_name="c", subcore_axis_name="s")
    n_nodes_w = _GPW * _NPER
    n_edges_w = _GPW * _EPER
    acc_w = _GPW * _NPER * _NPER
    chunks_g = _EPER // _L

    @functools.partial(
        pl.kernel,
        mesh=mesh,
        compiler_params=pltpu.CompilerParams(needs_layout_passes=False),
        out_type=jax.ShapeDtypeStruct((_NG * _NPER * _NPER,), jnp.float32),
        scratch_types=[
            pltpu.VMEM((n_nodes_w,), jnp.float32),
            pltpu.VMEM((n_nodes_w,), jnp.float32),
            pltpu.VMEM((n_edges_w,), jnp.int32),
            pltpu.VMEM((n_edges_w,), jnp.int32),
            pltpu.VMEM((_EPER,), jnp.float32),
            pltpu.VMEM((_EPER,), jnp.int32),
            pltpu.VMEM((acc_w,), jnp.float32),
        ],
    )
    def edge_kernel(hs_hbm, hd_hbm, src_hbm, dst_hbm, out_hbm,
                    hs_v, hd_v, src_v, dst_v, e_v, pidx_v, acc_v):
        wid = lax.axis_index("s") * _NC + lax.axis_index("c")
        nbase = pl.multiple_of(wid * n_nodes_w, n_nodes_w)
        ebase = pl.multiple_of(wid * n_edges_w, n_edges_w)
        obase = pl.multiple_of(wid * acc_w, acc_w)
        pltpu.sync_copy(hs_hbm.at[pl.ds(nbase, n_nodes_w)], hs_v)
        pltpu.sync_copy(hd_hbm.at[pl.ds(nbase, n_nodes_w)], hd_v)
        pltpu.sync_copy(src_hbm.at[pl.ds(ebase, n_edges_w)], src_v)
        pltpu.sync_copy(dst_hbm.at[pl.ds(ebase, n_edges_w)], dst_v)

        def zero(j, _):
            acc_v[pl.ds(j * _L, _L)] = jnp.zeros((_L,), jnp.float32)
            return 0
        lax.fori_loop(0, acc_w // _L, zero, 0)

        for gi in range(_GPW):
            goff = gi * _EPER

            def pass1(j, m):
                off = goff + j * _L
                s = src_v[pl.ds(off, _L)] - nbase
                d = dst_v[pl.ds(off, _L)] - nbase
                t = plsc.load_gather(hs_v, [s]) + plsc.load_gather(hd_v, [d])
                e = jnp.maximum(t, 0.2 * t)
                e_v[pl.ds(j * _L, _L)] = e
                sl = s - gi * _NPER
                dl = d - gi * _NPER
                pidx_v[pl.ds(j * _L, _L)] = (gi * _NPER + dl) * _NPER + sl
                return jnp.maximum(m, e)

            m = lax.fori_loop(0, chunks_g, pass1,
                              jnp.full((_L,), -3.0e38, jnp.float32))
            mg = jnp.max(m)

            def pass2(j, _):
                ex = jnp.exp(e_v[pl.ds(j * _L, _L)] - mg)
                plsc.addupdate_scatter(acc_v, [pidx_v[pl.ds(j * _L, _L)]], ex)
                return 0
            lax.fori_loop(0, chunks_g, pass2, 0)

        pltpu.sync_copy(acc_v, out_hbm.at[pl.ds(obase, acc_w)])

    return edge_kernel


_GPB = 8


def _attn_body(a_ref, h_ref, q_ref, wq_ref, wkt_ref, wv_ref, o_ref):
    qh = jnp.dot(q_ref[...], wq_ref[...], preferred_element_type=jnp.float32)
    wkq = jnp.dot(qh, wkt_ref[...], preferred_element_type=jnp.float32)
    scale = 1.0 / (float(_H) ** 0.5)
    rows = []
    for i in range(_GPB):
        m_raw = a_ref[i]
        rs = jnp.sum(m_raw, axis=1, keepdims=True)
        an = m_raw / jnp.where(rs == 0.0, 1.0, rs)
        hg = h_ref[i]
        ft = jnp.dot(an, hg, preferred_element_type=jnp.float32)
        s = jnp.sum(ft * wkq[i][None, :], axis=1, keepdims=True) * scale
        p = jnp.exp(s - jnp.max(s, axis=0, keepdims=True))
        p = p / jnp.sum(p, axis=0, keepdims=True)
        rows.append(jnp.sum(ft * p, axis=0, keepdims=True))
    ftp = jnp.concatenate(rows, axis=0)
    o_ref[...] = jnp.dot(ftp, wv_ref[...], preferred_element_type=jnp.float32)


_TB = 128


def _scatter_body(s_ref, e_ref, ao_ref, o_ref):
    tb = pl.program_id(1)
    tvals = tb * _TB + lax.broadcasted_iota(jnp.int32, (_TB, _G), 0)
    inside = (tvals >= s_ref[0]) & (tvals <= e_ref[0])
    insf = inside.astype(jnp.float32)
    ii = lax.broadcasted_iota(jnp.int32, (_G, _G), 0)
    jj = lax.broadcasted_iota(jnp.int32, (_G, _G), 1)
    tri = (ii <= jj).astype(jnp.float32)
    csum = jnp.dot(insf, tri, preferred_element_type=jnp.float32)
    sel = jnp.where(inside & (csum == 1.0), 1.0, 0.0)
    o_ref[0] = jnp.dot(sel, ao_ref[0], preferred_element_type=jnp.float32)


def kernel(node_feat, node_type_feat, query, edge_index, spans, seq_len,
           W, a_src, a_dst, Wq, Wk, Wv):
    del seq_len
    n_nodes, dn = node_feat.shape
    dt = node_type_feat.shape[1]
    rows_blk = n_nodes // 16

    a2 = jnp.pad(jnp.stack([a_src, a_dst], axis=-1), ((0, 0), (0, 6)))

    h, hsd = pl.pallas_call(
        _proj_body,
        grid=(16,),
        in_specs=[
            pl.BlockSpec((rows_blk, dn), lambda i: (i, 0)),
            pl.BlockSpec((rows_blk, dt), lambda i: (i, 0)),
            pl.BlockSpec((dn + dt, _H), lambda i: (0, 0)),
            pl.BlockSpec((dn + dt, 8), lambda i: (0, 0)),
        ],
        out_specs=[
            pl.BlockSpec((rows_blk, _H), lambda i: (i, 0)),
            pl.BlockSpec((rows_blk, 8), lambda i: (i, 0)),
        ],
        out_shape=[
            jax.ShapeDtypeStruct((n_nodes, _H), jnp.float32),
            jax.ShapeDtypeStruct((n_nodes, 8), jnp.float32),
        ],
    )(node_feat, node_type_feat, W, a2)

    hs = hsd[:, 0]
    hd = hsd[:, 1]
    src = edge_index[0]
    dst = edge_index[1]

    acc = _make_edge_kernel()(hs, hd, src, dst)
    a3 = acc.reshape(_NG, _NPER, _NPER)

    h3 = h.reshape(_NG, _NPER, _H)
    q2 = query.reshape(_NG, _H)
    ao = pl.pallas_call(
        _attn_body,
        grid=(_NG // _GPB,),
        in_specs=[
            pl.BlockSpec((_GPB, _NPER, _NPER), lambda i: (i, 0, 0)),
            pl.BlockSpec((_GPB, _NPER, _H), lambda i: (i, 0, 0)),
            pl.BlockSpec((_GPB, _H), lambda i: (i, 0)),
            pl.BlockSpec((_H, _H), lambda i: (0, 0)),
            pl.BlockSpec((_H, _H), lambda i: (0, 0)),
            pl.BlockSpec((_H, _H), lambda i: (0, 0)),
        ],
        out_specs=pl.BlockSpec((_GPB, _H), lambda i: (i, 0)),
        out_shape=jax.ShapeDtypeStruct((_NG, _H), jnp.float32),
    )(a3, h3, q2, Wq, Wk.T, Wv)

    ao3 = ao.reshape(_B, _G, _H)
    starts3 = spans[:, :, 0].reshape(_B, 1, _G)
    ends3 = spans[:, :, 1].reshape(_B, 1, _G)
    out = pl.pallas_call(
        _scatter_body,
        grid=(_B, _S // _TB),
        in_specs=[
            pl.BlockSpec((1, 1, _G), lambda b, t: (b, 0, 0)),
            pl.BlockSpec((1, 1, _G), lambda b, t: (b, 0, 0)),
            pl.BlockSpec((1, _G, _H), lambda b, t: (b, 0, 0)),
        ],
        out_specs=pl.BlockSpec((1, _TB, _H), lambda b, t: (b, t, 0)),
        out_shape=jax.ShapeDtypeStruct((_B, _S, _H), jnp.float32),
    )(starts3, ends3, ao3)
    return out


# speedup vs baseline: 60.2255x; 1.9525x over previous
import functools

import jax
import jax.numpy as jnp
from jax import lax
from jax.experimental import pallas as pl
from jax.experimental.pallas import tpu as pltpu
from jax.experimental.pallas import tpu_sc as plsc

_B, _G, _NPER, _EPER, _H, _S = 8, 16, 64, 1024, 512, 2048
_NG = _B * _G
_NC, _NS = 2, 16
_NW = _NC * _NS
_GPW = _NG // _NW
_L = 16


def _proj_body(nf_ref, nt_ref, w_ref, asrc_ref, adst_ref, h_ref, hsd_ref):
    dn = nf_ref.shape[1]
    h = jnp.dot(nf_ref[...], w_ref[:dn, :], preferred_element_type=jnp.float32)
    h = h + jnp.dot(nt_ref[...], w_ref[dn:, :], preferred_element_type=jnp.float32)
    h_ref[...] = h
    hs = jnp.sum(h * asrc_ref[...], axis=1)
    hd = jnp.sum(h * adst_ref[...], axis=1)
    hsd_ref[...] = jnp.stack([hs, hd], axis=0)


@functools.cache
def _make_edge_kernel():
    mesh = plsc.VectorSubcoreMesh(core_axis_name="c", subcore_axis_name="s")
    n_nodes_w = _GPW * _NPER
    n_edges_w = _GPW * _EPER
    acc_w = _GPW * _NPER * _NPER
    chunks_g = _EPER // _L

    @functools.partial(
        pl.kernel,
        mesh=mesh,
        compiler_params=pltpu.CompilerParams(needs_layout_passes=False),
        out_type=jax.ShapeDtypeStruct((_NG * _NPER * _NPER,), jnp.float32),
        scratch_types=[
            pltpu.VMEM((n_nodes_w,), jnp.float32),
            pltpu.VMEM((n_nodes_w,), jnp.float32),
            pltpu.VMEM((n_edges_w,), jnp.int32),
            pltpu.VMEM((n_edges_w,), jnp.int32),
            pltpu.VMEM((_EPER,), jnp.float32),
            pltpu.VMEM((_EPER,), jnp.int32),
            pltpu.VMEM((acc_w,), jnp.float32),
        ],
    )
    def edge_kernel(hsd_hbm, ei_hbm, out_hbm,
                    hs_v, hd_v, src_v, dst_v, e_v, pidx_v, acc_v):
        wid = lax.axis_index("s") * _NC + lax.axis_index("c")
        nbase = pl.multiple_of(wid * n_nodes_w, n_nodes_w)
        ebase = pl.multiple_of(wid * n_edges_w, n_edges_w)
        obase = pl.multiple_of(wid * acc_w, acc_w)
        pltpu.sync_copy(hsd_hbm.at[0, pl.ds(nbase, n_nodes_w)], hs_v)
        pltpu.sync_copy(hsd_hbm.at[1, pl.ds(nbase, n_nodes_w)], hd_v)
        pltpu.sync_copy(ei_hbm.at[0, pl.ds(ebase, n_edges_w)], src_v)
        pltpu.sync_copy(ei_hbm.at[1, pl.ds(ebase, n_edges_w)], dst_v)

        def zero(j, _):
            acc_v[pl.ds(j * _L, _L)] = jnp.zeros((_L,), jnp.float32)
            return 0
        lax.fori_loop(0, acc_w // _L, zero, 0)

        for gi in range(_GPW):
            goff = gi * _EPER

            def pass1(j, m):
                off = goff + j * _L
                s = src_v[pl.ds(off, _L)] - nbase
                d = dst_v[pl.ds(off, _L)] - nbase
                t = plsc.load_gather(hs_v, [s]) + plsc.load_gather(hd_v, [d])
                e = jnp.maximum(t, 0.2 * t)
                e_v[pl.ds(j * _L, _L)] = e
                sl = s - gi * _NPER
                dl = d - gi * _NPER
                pidx_v[pl.ds(j * _L, _L)] = (gi * _NPER + dl) * _NPER + sl
                return jnp.maximum(m, e)

            m = lax.fori_loop(0, chunks_g, pass1,
                              jnp.full((_L,), -3.0e38, jnp.float32))
            mg = jnp.max(m)

            def pass2(j, _):
                ex = jnp.exp(e_v[pl.ds(j * _L, _L)] - mg)
                plsc.addupdate_scatter(acc_v, [pidx_v[pl.ds(j * _L, _L)]], ex)
                return 0
            lax.fori_loop(0, chunks_g, pass2, 0)

        pltpu.sync_copy(acc_v, out_hbm.at[pl.ds(obase, acc_w)])

    return edge_kernel


def _attn_out_body(a_ref, h_ref, q_ref, s_ref, e_ref,
                   wq_ref, wkt_ref, wv_ref, o_ref):
    qh = jnp.dot(q_ref[0], wq_ref[...], preferred_element_type=jnp.float32)
    wkq = jnp.dot(qh, wkt_ref[...], preferred_element_type=jnp.float32)
    scale = 1.0 / (float(_H) ** 0.5)
    rows = []
    for i in range(_G):
        m_raw = a_ref[0, i]
        rs = jnp.sum(m_raw, axis=1, keepdims=True)
        an = m_raw / jnp.where(rs == 0.0, 1.0, rs)
        hg = h_ref[0, i]
        ft = jnp.dot(an, hg, preferred_element_type=jnp.float32)
        s = jnp.sum(ft * wkq[i][None, :], axis=1, keepdims=True) * scale
        p = jnp.exp(s - jnp.max(s, axis=0, keepdims=True))
        p = p / jnp.sum(p, axis=0, keepdims=True)
        rows.append(jnp.sum(ft * p, axis=0, keepdims=True))
    ftp = jnp.concatenate(rows, axis=0)
    ao = jnp.dot(ftp, wv_ref[...], preferred_element_type=jnp.float32)

    tvals = lax.broadcasted_iota(jnp.int32, (_S, _G), 0)
    inside = (tvals >= s_ref[0]) & (tvals <= e_ref[0])
    insf = inside.astype(jnp.float32)
    ii = lax.broadcasted_iota(jnp.int32, (_G, _G), 0)
    jj = lax.broadcasted_iota(jnp.int32, (_G, _G), 1)
    tri = (ii <= jj).astype(jnp.float32)
    csum = jnp.dot(insf, tri, preferred_element_type=jnp.float32)
    sel = jnp.where(inside & (csum == 1.0), 1.0, 0.0)
    o_ref[0] = jnp.dot(sel, ao, preferred_element_type=jnp.float32)


def kernel(node_feat, node_type_feat, query, edge_index, spans, seq_len,
           W, a_src, a_dst, Wq, Wk, Wv):
    del seq_len
    n_nodes, dn = node_feat.shape
    dt = node_type_feat.shape[1]
    rows_blk = n_nodes // 16

    h, hsd = pl.pallas_call(
        _proj_body,
        grid=(16,),
        in_specs=[
            pl.BlockSpec((rows_blk, dn), lambda i: (i, 0)),
            pl.BlockSpec((rows_blk, dt), lambda i: (i, 0)),
            pl.BlockSpec((dn + dt, _H), lambda i: (0, 0)),
            pl.BlockSpec((1, _H), lambda i: (0, 0)),
            pl.BlockSpec((1, _H), lambda i: (0, 0)),
        ],
        out_specs=[
            pl.BlockSpec((rows_blk, _H), lambda i: (i, 0)),
            pl.BlockSpec((2, rows_blk), lambda i: (0, i)),
        ],
        out_shape=[
            jax.ShapeDtypeStruct((n_nodes, _H), jnp.float32),
            jax.ShapeDtypeStruct((2, n_nodes), jnp.float32),
        ],
    )(node_feat, node_type_feat, W,
      a_src.reshape(1, _H), a_dst.reshape(1, _H))

    acc = _make_edge_kernel()(hsd, edge_index)
    a4 = acc.reshape(_B, _G, _NPER, _NPER)

    h4 = h.reshape(_B, _G, _NPER, _H)
    q3 = query
    starts3 = spans[:, :, 0].reshape(_B, 1, _G)
    ends3 = spans[:, :, 1].reshape(_B, 1, _G)
    out = pl.pallas_call(
        _attn_out_body,
        grid=(_B,),
        in_specs=[
            pl.BlockSpec((1, _G, _NPER, _NPER), lambda b: (b, 0, 0, 0)),
            pl.BlockSpec((1, _G, _NPER, _H), lambda b: (b, 0, 0, 0)),
            pl.BlockSpec((1, _G, _H), lambda b: (b, 0, 0)),
            pl.BlockSpec((1, 1, _G), lambda b: (b, 0, 0)),
            pl.BlockSpec((1, 1, _G), lambda b: (b, 0, 0)),
            pl.BlockSpec((_H, _H), lambda b: (0, 0)),
            pl.BlockSpec((_H, _H), lambda b: (0, 0)),
            pl.BlockSpec((_H, _H), lambda b: (0, 0)),
        ],
        out_specs=pl.BlockSpec((1, _S, _H), lambda b: (b, 0, 0)),
        out_shape=jax.ShapeDtypeStruct((_B, _S, _H), jnp.float32),
    )(a4, h4, q3, starts3, ends3, Wq, Wk.T, Wv)
    return out
